# baseline (speedup 1.0000x reference)
import functools

import numpy as np
import jax
import jax.numpy as jnp
from jax import lax
from jax.experimental import pallas as pl
from jax.experimental.pallas import tpu as pltpu
from jax.experimental.pallas import tpu_sc as plsc

_N = 401408
_D = 96
_DP = 128
_NC = 2
_NS = 16
_NW = _NC * _NS
_CH = 128
_RW = _N // _NW
_NCH_W = _RW // _CH


def _compute_permutation():
    with jax.set_mesh(None), jax.ensure_compile_time_eval():
        p = jax.random.permutation(jax.random.key(42), _N)
        return np.asarray(p, dtype=np.int32)


_PERM = _compute_permutation()

_RA = 2048


def _tc_stage(feat_flat, mask_col):
    def body(f_ref, m_ref, h_ref):
        bg = (1.0 - m_ref[...]) * f_ref[...]
        h_ref[:, : _D] = bg

    return pl.pallas_call(
        body,
        grid=(_N // _RA,),
        in_specs=[
            pl.BlockSpec((_RA, _D), lambda i: (i, 0)),
            pl.BlockSpec((_RA, 1), lambda i: (i, 0)),
        ],
        out_specs=pl.BlockSpec((_RA, _DP), lambda i: (i, 0)),
        out_shape=jax.ShapeDtypeStruct((_N, _DP), jnp.float32),
    )(feat_flat, mask_col)


def _sc_gather(h, perm):
    mesh = plsc.VectorSubcoreMesh(core_axis_name="c", subcore_axis_name="s")

    @functools.partial(
        pl.kernel,
        out_type=jax.ShapeDtypeStruct((_N, _DP), jnp.float32),
        mesh=mesh,
        scratch_types=[
            pltpu.VMEM((_RW,), jnp.int32),
            pltpu.VMEM((_CH, _DP), jnp.float32),
            pltpu.VMEM((_CH, _DP), jnp.float32),
            pltpu.SemaphoreType.DMA,
            pltpu.SemaphoreType.DMA,
        ],
    )
    def k(h_hbm, perm_hbm, g_hbm, idx_v, buf0, buf1, sem0, sem1):
        wid = lax.axis_index("c") * _NS + lax.axis_index("s")
        rbase = wid * _RW
        pltpu.sync_copy(perm_hbm.at[pl.ds(rbase, _RW)], idx_v)

        def body(jj, carry):
            j0 = jj * 2
            idx0 = idx_v.at[pl.ds(j0 * _CH, _CH)]
            idx1 = idx_v.at[pl.ds((j0 + 1) * _CH, _CH)]
            cp0 = pltpu.async_copy(h_hbm.at[idx0], buf0, sem0)
            cp1 = pltpu.async_copy(h_hbm.at[idx1], buf1, sem1)
            row0 = rbase + j0 * _CH
            cp0.wait()
            pltpu.sync_copy(buf0, g_hbm.at[pl.ds(row0, _CH)])
            cp1.wait()
            pltpu.sync_copy(buf1, g_hbm.at[pl.ds(row0 + _CH, _CH)])
            return carry

        lax.fori_loop(0, _NCH_W // 2, body, 0)

    return k(h, perm)


def _tc_combine(feat_flat, mask_col, g):
    def body(f_ref, m_ref, g_ref, o_ref):
        m = m_ref[...]
        o_ref[...] = m * f_ref[...] + (1.0 - m) * g_ref[:, : _D]

    return pl.pallas_call(
        body,
        grid=(_N // _RA,),
        in_specs=[
            pl.BlockSpec((_RA, _D), lambda i: (i, 0)),
            pl.BlockSpec((_RA, 1), lambda i: (i, 0)),
            pl.BlockSpec((_RA, _DP), lambda i: (i, 0)),
        ],
        out_specs=pl.BlockSpec((_RA, _D), lambda i: (i, 0)),
        out_shape=jax.ShapeDtypeStruct((_N, _D), jnp.float32),
    )(feat_flat, mask_col, g)


def kernel(feat, mask):
    feat_flat = feat.reshape(_N, _D)
    mask_col = mask.reshape(_N, 1)
    perm = jnp.asarray(_PERM)
    h = _tc_stage(feat_flat, mask_col)
    g = _sc_gather(h, perm)
    out = _tc_combine(feat_flat, mask_col, g)
    return out.reshape(feat.shape)
